# Initial kernel scaffold; baseline (speedup 1.0000x reference)
import jax
import jax.numpy as jnp
from jax.experimental import pallas as pl


def kernel(input, edge_index, edge_weight):
    raise NotImplementedError("write your pallas kernel here")


# speedup vs baseline: 4.0826x; 4.0826x over previous
import functools

import jax
import jax.numpy as jnp
from jax import lax
from jax.experimental import pallas as pl
from jax.experimental.pallas import tpu as pltpu
from jax.experimental.pallas import tpu_sc as plsc

_NC = 2
_NS = 16
_L = 16
_NW = _NC * _NS


def _sc_body(n, d, e_t, c, zr, x_hbm, src_hbm, dst_hbm, w_hbm, out_hbm,
             acc, srcb, dstb, wb, rows, zbuf, sem):
    cid = lax.axis_index("c")
    sid = lax.axis_index("s")
    wid = cid * _NS + sid
    nch = n // zr

    def zrow(r, carry):
        for j in range(d // _L):
            zbuf[r, pl.ds(j * _L, _L)] = jnp.zeros((_L,), jnp.float32)
        return carry
    lax.fori_loop(0, zr, zrow, 0)
    for q in range((nch + _NS - 1) // _NS):
        idx = sid + _NS * q

        @pl.when(idx < nch)
        def _():
            pltpu.sync_copy(zbuf, acc.at[pl.ds(idx * zr, zr)])
    plsc.subcore_barrier()

    base = wid * e_t

    def chunk(k, carry):
        off = base + k * c
        pltpu.sync_copy(src_hbm.at[pl.ds(off, c)], srcb)
        pltpu.sync_copy(dst_hbm.at[pl.ds(off, c)], dstb)
        pltpu.sync_copy(w_hbm.at[pl.ds(off, c)], wb)
        pltpu.async_copy(x_hbm.at[srcb], rows, sem).wait()

        def scale(i, icarry):
            ws = plsc.load_gather(wb, [jnp.full((_L,), i, jnp.int32)])
            for j in range(d // _L):
                rows[i, pl.ds(j * _L, _L)] = rows[i, pl.ds(j * _L, _L)] * ws
            return icarry
        lax.fori_loop(0, c, scale, 0)

        pltpu.sync_copy(rows, acc.at[dstb], add=True)
        return carry
    lax.fori_loop(0, e_t // c, chunk, 0)
    plsc.subcore_barrier()

    for q in range((nch + _NS - 1) // _NS):
        idx = sid + _NS * q

        @pl.when(idx < nch)
        def _():
            r0 = idx * zr
            pltpu.sync_copy(acc.at[pl.ds(r0, zr)],
                            out_hbm.at[cid, pl.ds(r0, zr)])


def _combine_body(p_ref, o_ref):
    o_ref[...] = p_ref[0] + p_ref[1]


@jax.jit
def kernel(input, edge_index, edge_weight):
    n, d = input.shape
    e = edge_index.shape[1]
    assert e % _NW == 0 and n % _NS == 0 and d % _L == 0
    e_t = e // _NW
    c = 80
    assert e_t % c == 0
    zr = 200
    assert n % zr == 0 and zr % 8 == 0

    mesh = plsc.VectorSubcoreMesh(core_axis_name="c", subcore_axis_name="s",
                                  num_cores=_NC, num_subcores=_NS)
    partial = pl.kernel(
        functools.partial(_sc_body, n, d, e_t, c, zr),
        out_type=jax.ShapeDtypeStruct((_NC, n, d), jnp.float32),
        mesh=mesh,
        compiler_params=pltpu.CompilerParams(needs_layout_passes=False),
        scratch_types=[
            pltpu.MemorySpace.VMEM_SHARED((n, d), jnp.float32),
            pltpu.VMEM((c,), jnp.int32),
            pltpu.VMEM((c,), jnp.int32),
            pltpu.VMEM((c,), jnp.float32),
            pltpu.VMEM((c, d), jnp.float32),
            pltpu.VMEM((zr, d), jnp.float32),
            pltpu.SemaphoreType.DMA,
        ],
    )(input, edge_index[1], edge_index[0], edge_weight)

    r = 2000
    return pl.pallas_call(
        _combine_body,
        grid=(n // r,),
        in_specs=[pl.BlockSpec((2, r, d), lambda i: (0, i, 0))],
        out_specs=pl.BlockSpec((r, d), lambda i: (i, 0)),
        out_shape=jax.ShapeDtypeStruct((n, d), jnp.float32),
    )(partial)
